# baseline (speedup 1.0000x reference)
import functools

import jax
import jax.numpy as jnp
from jax import lax
from jax.experimental import pallas as pl
from jax.experimental.pallas import tpu as pltpu
from jax.experimental.pallas import tpu_sc as plsc

_V = 10242
_KN = 9
_KS = 9
_BS = 8
_C = 32
_BC = _BS * _C
_VB = 512
_VPAD = 10752
_NT = 3
_VT = _VPAD // _NT
_NB = _VT // _VB
_NC = 2
_NS = 16
_NW = _NC * _NS
_ROWS = _KN * _VT
_PER_W = _ROWS // _NW
_CH = 168
_NCH = _PER_W // _CH


def _sc_gather(table, idx_flat):
    mesh = plsc.VectorSubcoreMesh(core_axis_name="c", subcore_axis_name="s")

    @functools.partial(
        pl.kernel,
        mesh=mesh,
        out_type=jax.ShapeDtypeStruct((_ROWS, _BC // 2), jnp.int32),
        scratch_types=[
            pltpu.VMEM((_PER_W,), jnp.int32),
            pltpu.VMEM((_CH, _BC // 2), jnp.int32),
            pltpu.VMEM((_CH, _BC // 2), jnp.int32),
            pltpu.VMEM((320, _BC // 2), jnp.int32),
            pltpu.SemaphoreType.DMA,
            pltpu.SemaphoreType.DMA,
            pltpu.SemaphoreType.DMA,
            pltpu.SemaphoreType.DMA,
        ],
    )
    def gather_kernel(table_hbm, idx_hbm, out_hbm,
                      idx_all, r0, r1, warm_v, sg0, sg1, ss0, ss1):
        wid = lax.axis_index("s") * _NC + lax.axis_index("c")
        wbase = wid * _PER_W
        pltpu.sync_copy(table_hbm.at[pl.ds(wid * 320, 320)], warm_v)
        pltpu.sync_copy(idx_hbm.at[pl.ds(wbase, _PER_W)], idx_all)
        rows, sg, ss = (r0, r1), (sg0, sg1), (ss0, ss1)
        scatters = [None, None]
        for i in range(_NCH):
            b = i % 2
            if scatters[b] is not None:
                scatters[b].wait()
            g = pltpu.async_copy(
                table_hbm.at[idx_all.at[pl.ds(i * _CH, _CH)]], rows[b], sg[b])
            g.wait()
            scatters[b] = pltpu.async_copy(
                rows[b], out_hbm.at[pl.ds(wbase + i * _CH, _CH)], ss[b])
        for b in range(2):
            if scatters[b] is not None:
                scatters[b].wait()

    return gather_kernel(table, idx_flat)


def _tc_body(g_ref, itp_ref, w3_ref, ones_ref, bias_ref, out_ref):
    gf = []
    for k in range(_KN):
        gi = g_ref[k]
        lo = lax.bitcast_convert_type(gi << 16, jnp.float32)
        hi = lax.bitcast_convert_type(gi & jnp.int32(-65536), jnp.float32)
        gf.append(jnp.concatenate([lo, hi], axis=1))
    nzf = (gf[0] != 0.0).astype(jnp.float32)
    for k in range(1, _KN):
        nzf += (gf[k] != 0.0).astype(jnp.float32)
    cnt = jnp.dot(nzf, ones_ref[...], preferred_element_type=jnp.float32)
    acc = jnp.zeros((_VB, _BC), jnp.float32)
    for s in range(_KS):
        xs = gf[0] * itp_ref[:, s:s + 1]
        for k in range(1, _KN):
            xs = xs + gf[k] * itp_ref[:, k * _KS + s:k * _KS + s + 1]
        acc = acc + jnp.dot(xs, w3_ref[s], preferred_element_type=jnp.float32)
    acc = acc + bias_ref[0][None, :]
    out_ref[...] = jnp.where(cnt > 0.0, acc, 0.0)


def _tc_compute(g, itp_r, w3, ones_bd, bias_bc):
    return pl.pallas_call(
        _tc_body,
        grid=(_NB,),
        in_specs=[
            pl.BlockSpec((_KN, _VB, _BC // 2), lambda i: (0, i, 0)),
            pl.BlockSpec((_VB, _KN * _KS), lambda i: (i, 0)),
            pl.BlockSpec((_KS, _BC, _BC), lambda i: (0, 0, 0)),
            pl.BlockSpec((_BC, _BC), lambda i: (0, 0)),
            pl.BlockSpec((1, _BC), lambda i: (0, 0)),
        ],
        out_specs=pl.BlockSpec((_VB, _BC), lambda i: (i, 0)),
        out_shape=jax.ShapeDtypeStruct((_VT, _BC), jnp.float32),
    )(g, itp_r, w3, ones_bd, bias_bc)


def kernel(tensor, index, itp_mat, conv_weight, conv_bias):
    bs, c, v_num = tensor.shape
    tbl = jnp.transpose(tensor, (2, 0, 1)).reshape(v_num, bs * c)
    d16 = lax.bitcast_convert_type(tbl.astype(jnp.bfloat16), jnp.uint16)
    w_lo = d16[:, :_BC // 2].astype(jnp.uint32)
    w_hi = d16[:, _BC // 2:].astype(jnp.uint32)
    table = lax.bitcast_convert_type(w_lo | (w_hi << 16), jnp.int32)
    idx_pad = jnp.pad(index.astype(jnp.int32), ((0, _VPAD - v_num), (0, 0)))

    itp_pad = jnp.pad(itp_mat, ((0, _VPAD - v_num), (0, 0), (0, 0)))
    itp_r = itp_pad.reshape(_VPAD, _KN * _KS)

    w = conv_weight[:, :, 0, :]
    eye8 = jnp.eye(_BS, dtype=jnp.float32)
    w3 = jnp.einsum("ocs,de->sdceo", w, eye8).reshape(_KS, _BC, _BC)
    ones_bd = jnp.kron(eye8, jnp.ones((_C, _C), jnp.float32))
    bias_bc = jnp.tile(conv_bias, _BS).reshape(1, _BC)

    outs = []
    for t in range(_NT):
        idx_t = jnp.transpose(
            idx_pad[t * _VT:(t + 1) * _VT], (1, 0)).reshape(-1)
        g_t = _sc_gather(table, idx_t).reshape(_KN, _VT, _BC // 2)
        outs.append(g_t)
    return outs


# speedup vs baseline: 1.0503x; 1.0503x over previous
import functools

import jax
import jax.numpy as jnp
from jax import lax
from jax.experimental import pallas as pl
from jax.experimental.pallas import tpu as pltpu
from jax.experimental.pallas import tpu_sc as plsc

_V = 10242
_KN = 9
_KS = 9
_BS = 8
_C = 32
_BC = _BS * _C
_VB = 512
_VPAD = 10752
_NT = 3
_VT = _VPAD // _NT
_NB = _VT // _VB
_NC = 2
_NS = 16
_NW = _NC * _NS
_ROWS = _KN * _VT
_PER_W = _ROWS // _NW
_CH = 168
_NCH = _PER_W // _CH


def _sc_gather(table, idx_flat):
    mesh = plsc.VectorSubcoreMesh(core_axis_name="c", subcore_axis_name="s")

    @functools.partial(
        pl.kernel,
        mesh=mesh,
        out_type=jax.ShapeDtypeStruct((_ROWS, _BC // 2), jnp.int32),
        scratch_types=[
            pltpu.VMEM((_PER_W,), jnp.int32),
            pltpu.VMEM((_CH, _BC // 2), jnp.int32),
            pltpu.VMEM((_CH, _BC // 2), jnp.int32),
            pltpu.VMEM_SHARED((_V, _BC // 2), jnp.int32),
            pltpu.SemaphoreType.DMA,
            pltpu.SemaphoreType.DMA,
            pltpu.SemaphoreType.DMA,
            pltpu.SemaphoreType.DMA,
        ],
    )
    def gather_kernel(table_hbm, idx_hbm, out_hbm,
                      idx_all, r0, r1, tbl_s, sg0, sg1, ss0, ss1):
        sid = lax.axis_index("s")
        wid = sid * _NC + lax.axis_index("c")
        wbase = wid * _PER_W
        pltpu.sync_copy(table_hbm.at[pl.ds(sid * 640, 640)],
                        tbl_s.at[pl.ds(sid * 640, 640)])

        @pl.when(sid == 0)
        def _():
            pltpu.sync_copy(table_hbm.at[pl.ds(10240, 2)],
                            tbl_s.at[pl.ds(10240, 2)])

        pltpu.sync_copy(idx_hbm.at[pl.ds(wbase, _PER_W)], idx_all)
        plsc.subcore_barrier()
        rows, sg, ss = (r0, r1), (sg0, sg1), (ss0, ss1)
        scatters = [None, None]
        for i in range(_NCH):
            b = i % 2
            if scatters[b] is not None:
                scatters[b].wait()
            g = pltpu.async_copy(
                tbl_s.at[idx_all.at[pl.ds(i * _CH, _CH)]], rows[b], sg[b])
            g.wait()
            scatters[b] = pltpu.async_copy(
                rows[b], out_hbm.at[pl.ds(wbase + i * _CH, _CH)], ss[b])
        for b in range(2):
            if scatters[b] is not None:
                scatters[b].wait()

    return gather_kernel(table, idx_flat)


def _tc_body(g_ref, itp_ref, w3_ref, ones_ref, bias_ref, out_ref):
    gf = []
    for k in range(_KN):
        gi = g_ref[k]
        lo = lax.bitcast_convert_type(gi << 16, jnp.float32)
        hi = lax.bitcast_convert_type(gi & jnp.int32(-65536), jnp.float32)
        gf.append(jnp.concatenate([lo, hi], axis=1))
    nzf = (gf[0] != 0.0).astype(jnp.float32)
    for k in range(1, _KN):
        nzf += (gf[k] != 0.0).astype(jnp.float32)
    cnt = jnp.dot(nzf, ones_ref[...], preferred_element_type=jnp.float32)
    acc = jnp.zeros((_VB, _BC), jnp.float32)
    for s in range(_KS):
        xs = gf[0] * itp_ref[:, s:s + 1]
        for k in range(1, _KN):
            xs = xs + gf[k] * itp_ref[:, k * _KS + s:k * _KS + s + 1]
        acc = acc + jnp.dot(xs, w3_ref[s], preferred_element_type=jnp.float32)
    acc = acc + bias_ref[0][None, :]
    out_ref[...] = jnp.where(cnt > 0.0, acc, 0.0)


def _tc_compute(g, itp_r, w3, ones_bd, bias_bc):
    return pl.pallas_call(
        _tc_body,
        grid=(_NB,),
        in_specs=[
            pl.BlockSpec((_KN, _VB, _BC // 2), lambda i: (0, i, 0)),
            pl.BlockSpec((_VB, _KN * _KS), lambda i: (i, 0)),
            pl.BlockSpec((_KS, _BC, _BC), lambda i: (0, 0, 0)),
            pl.BlockSpec((_BC, _BC), lambda i: (0, 0)),
            pl.BlockSpec((1, _BC), lambda i: (0, 0)),
        ],
        out_specs=pl.BlockSpec((_VB, _BC), lambda i: (i, 0)),
        out_shape=jax.ShapeDtypeStruct((_VT, _BC), jnp.float32),
    )(g, itp_r, w3, ones_bd, bias_bc)


def kernel(tensor, index, itp_mat, conv_weight, conv_bias):
    bs, c, v_num = tensor.shape
    tbl = jnp.transpose(tensor, (2, 0, 1)).reshape(v_num, bs * c)
    d16 = lax.bitcast_convert_type(tbl.astype(jnp.bfloat16), jnp.uint16)
    w_lo = d16[:, :_BC // 2].astype(jnp.uint32)
    w_hi = d16[:, _BC // 2:].astype(jnp.uint32)
    table = lax.bitcast_convert_type(w_lo | (w_hi << 16), jnp.int32)
    idx_pad = jnp.pad(index.astype(jnp.int32), ((0, _VPAD - v_num), (0, 0)))

    itp_pad = jnp.pad(itp_mat, ((0, _VPAD - v_num), (0, 0), (0, 0)))
    itp_r = itp_pad.reshape(_VPAD, _KN * _KS)

    w = conv_weight[:, :, 0, :]
    eye8 = jnp.eye(_BS, dtype=jnp.float32)
    w3 = jnp.einsum("ocs,de->sdceo", w, eye8).reshape(_KS, _BC, _BC)
    ones_bd = jnp.kron(eye8, jnp.ones((_C, _C), jnp.float32))
    bias_bc = jnp.tile(conv_bias, _BS).reshape(1, _BC)

    outs = []
    for t in range(_NT):
        idx_t = jnp.transpose(
            idx_pad[t * _VT:(t + 1) * _VT], (1, 0)).reshape(-1)
        g_t = _sc_gather(table, idx_t).reshape(_KN, _VT, _BC // 2)
        outs.append(_tc_compute(
            g_t, itp_r[t * _VT:(t + 1) * _VT], w3, ones_bd, bias_bc))
    out = jnp.concatenate(outs, axis=0)
    return jnp.transpose(out[:v_num], (1, 0)).reshape(bs, c, v_num)


# speedup vs baseline: 1.0829x; 1.0310x over previous
import functools

import jax
import jax.numpy as jnp
from jax import lax
from jax.experimental import pallas as pl
from jax.experimental.pallas import tpu as pltpu
from jax.experimental.pallas import tpu_sc as plsc

_V = 10242
_KN = 9
_KS = 9
_BS = 8
_C = 32
_BC = _BS * _C
_VB = 512
_VPAD = 10752
_NT = 3
_VT = _VPAD // _NT
_NB = _VT // _VB
_NC = 2
_NS = 16
_NW = _NC * _NS
_ROWS = _KN * _VT
_PER_W = _ROWS // _NW
_CH = 168
_NCH = _PER_W // _CH


def _sc_gather(table, idx_flat):
    mesh = plsc.VectorSubcoreMesh(core_axis_name="c", subcore_axis_name="s")

    @functools.partial(
        pl.kernel,
        mesh=mesh,
        out_type=jax.ShapeDtypeStruct((_ROWS, _BC // 2), jnp.int32),
        scratch_types=[
            pltpu.VMEM((_PER_W,), jnp.int32),
            pltpu.VMEM((_CH, _BC // 2), jnp.int32),
            pltpu.VMEM((_CH, _BC // 2), jnp.int32),
            pltpu.VMEM_SHARED((_V, _BC // 2), jnp.int32),
            pltpu.SemaphoreType.DMA,
            pltpu.SemaphoreType.DMA,
            pltpu.SemaphoreType.DMA,
            pltpu.SemaphoreType.DMA,
        ],
    )
    def gather_kernel(table_hbm, idx_hbm, out_hbm,
                      idx_all, r0, r1, tbl_s, sg0, sg1, ss0, ss1):
        sid = lax.axis_index("s")
        wid = sid * _NC + lax.axis_index("c")
        wbase = wid * _PER_W
        pltpu.sync_copy(table_hbm.at[pl.ds(sid * 640, 640)],
                        tbl_s.at[pl.ds(sid * 640, 640)])

        @pl.when(sid == 0)
        def _():
            pltpu.sync_copy(table_hbm.at[pl.ds(10240, 2)],
                            tbl_s.at[pl.ds(10240, 2)])

        pltpu.sync_copy(idx_hbm.at[pl.ds(wbase, _PER_W)], idx_all)
        plsc.subcore_barrier()
        rows, sg, ss = (r0, r1), (sg0, sg1), (ss0, ss1)
        scatters = [None, None]
        for i in range(_NCH):
            b = i % 2
            if scatters[b] is not None:
                scatters[b].wait()
            g = pltpu.async_copy(
                tbl_s.at[idx_all.at[pl.ds(i * _CH, _CH)]], rows[b], sg[b])
            g.wait()
            scatters[b] = pltpu.async_copy(
                rows[b], out_hbm.at[pl.ds(wbase + i * _CH, _CH)], ss[b])
        for b in range(2):
            if scatters[b] is not None:
                scatters[b].wait()

    return gather_kernel(table, idx_flat)


def _tc_body(g_ref, itp_ref, w3_ref, ones_ref, bias_ref, out_ref):
    gf = []
    for k in range(_KN):
        gi = g_ref[k]
        lo = lax.bitcast_convert_type(gi << 16, jnp.float32)
        hi = lax.bitcast_convert_type(gi & jnp.int32(-65536), jnp.float32)
        gf.append(jnp.concatenate([lo, hi], axis=1))
    nzf = (gf[0] != 0.0).astype(jnp.float32)
    for k in range(1, _KN):
        nzf += (gf[k] != 0.0).astype(jnp.float32)
    cnt = jnp.dot(nzf, ones_ref[...], preferred_element_type=jnp.float32)
    acc = jnp.zeros((_VB, _BC), jnp.float32)
    for s in range(_KS):
        xs = gf[0] * itp_ref[:, s:s + 1]
        for k in range(1, _KN):
            xs = xs + gf[k] * itp_ref[:, k * _KS + s:k * _KS + s + 1]
        acc = acc + jnp.dot(xs, w3_ref[s], preferred_element_type=jnp.float32)
    acc = acc + bias_ref[0][None, :]
    out_ref[...] = jnp.where(cnt > 0.0, acc, 0.0)


def _tc_compute(g, itp_r, w3, ones_bd, bias_bc):
    return pl.pallas_call(
        _tc_body,
        grid=(_NB,),
        in_specs=[
            pl.BlockSpec((_KN, _VB, _BC // 2), lambda i: (0, i, 0)),
            pl.BlockSpec((_VB, _KN * _KS), lambda i: (i, 0)),
            pl.BlockSpec((_KS, _BC, _BC), lambda i: (0, 0, 0)),
            pl.BlockSpec((_BC, _BC), lambda i: (0, 0)),
            pl.BlockSpec((1, _BC), lambda i: (0, 0)),
        ],
        out_specs=pl.BlockSpec((_VB, _BC), lambda i: (i, 0)),
        out_shape=jax.ShapeDtypeStruct((_VT, _BC), jnp.float32),
    )(g, itp_r, w3, ones_bd, bias_bc)


def kernel(tensor, index, itp_mat, conv_weight, conv_bias):
    bs, c, v_num = tensor.shape
    tbl = jnp.transpose(tensor, (2, 0, 1)).reshape(v_num, bs * c)
    d16 = lax.bitcast_convert_type(tbl.astype(jnp.bfloat16), jnp.uint16)
    w_lo = d16[:, :_BC // 2].astype(jnp.uint32)
    w_hi = d16[:, _BC // 2:].astype(jnp.uint32)
    table = lax.bitcast_convert_type(w_lo | (w_hi << 16), jnp.int32)
    idx_km = jnp.transpose(index.astype(jnp.int32), (1, 0))
    pad_v = _VPAD - v_num
    idx_ts = [
        idx_km[:, :_VT].reshape(-1),
        idx_km[:, _VT:2 * _VT].reshape(-1),
        jnp.pad(idx_km[:, 2 * _VT:], ((0, 0), (0, pad_v))).reshape(-1),
    ]
    itp_ts = [
        itp_mat[:_VT].reshape(_VT, _KN * _KS),
        itp_mat[_VT:2 * _VT].reshape(_VT, _KN * _KS),
        jnp.pad(itp_mat[2 * _VT:],
                ((0, pad_v), (0, 0), (0, 0))).reshape(_VT, _KN * _KS),
    ]

    w = conv_weight[:, :, 0, :]
    eye8 = jnp.eye(_BS, dtype=jnp.float32)
    w3 = jnp.einsum("ocs,de->sdceo", w, eye8).reshape(_KS, _BC, _BC)
    ones_bd = jnp.kron(eye8, jnp.ones((_C, _C), jnp.float32))
    bias_bc = jnp.tile(conv_bias, _BS).reshape(1, _BC)

    out = jnp.zeros((_VPAD, _BC), jnp.float32)
    for t in range(_NT):
        g_t = _sc_gather(table, idx_ts[t]).reshape(_KN, _VT, _BC // 2)
        o_t = _tc_compute(g_t, itp_ts[t], w3, ones_bd, bias_bc)
        out = lax.dynamic_update_slice(out, o_t, (t * _VT, 0))
    return jnp.transpose(out[:v_num], (1, 0)).reshape(bs, c, v_num)
